# baseline (speedup 1.0000x reference)
import functools

import jax
import jax.numpy as jnp
from jax import lax
from jax.experimental import pallas as pl
from jax.experimental.pallas import tpu as pltpu
from jax.experimental.pallas import tpu_sc as plsc

N = 10000
NPAD = 10240
D_IN = 128
D_H = 16
N_CLS = 7
E = 320000

NC = 2
NS = 16
NW = NC * NS
BV = 256
CH = 40
EPW = CH * BV
EPAD = NW * EPW
RPT = NPAD // NS
NB = 8
DEG_LAG = 8

_mesh = plsc.VectorSubcoreMesh(
    core_axis_name="c", subcore_axis_name="s", num_cores=NC, num_subcores=NS
)
_sc_params = pltpu.CompilerParams(use_tc_tiling_on_sc=False)


@functools.partial(
    pl.kernel,
    out_type=jax.ShapeDtypeStruct((NC, 2, NPAD), jnp.float32),
    mesh=_mesh,
    compiler_params=_sc_params,
    scratch_types=[
        pltpu.VMEM((CH, BV), jnp.int32),
        pltpu.VMEM((CH, BV), jnp.int32),
        pltpu.VMEM((BV,), jnp.float32),
        pltpu.VMEM_SHARED((NPAD,), jnp.float32),
        pltpu.VMEM_SHARED((NPAD,), jnp.float32),
        pltpu.SemaphoreType.DMA,
    ],
)
def _sc_degrees(src_h, dst_h, znode_h, out_h, src_v, dst_v, ones_v, dS_sh, dD_sh, sem):
    c = lax.axis_index("c")
    s = lax.axis_index("s")
    wid = s * NC + c
    for i in range(BV // 16):
        ones_v[pl.ds(i * 16, 16)] = jnp.ones((16,), jnp.float32)
    pltpu.sync_copy(znode_h.at[pl.ds(s * RPT, RPT)], dS_sh.at[pl.ds(s * RPT, RPT)])
    pltpu.sync_copy(znode_h.at[pl.ds(s * RPT, RPT)], dD_sh.at[pl.ds(s * RPT, RPT)])
    plsc.subcore_barrier()
    pltpu.sync_copy(src_h.at[pl.ds(wid * CH, CH)], src_v)
    pltpu.sync_copy(dst_h.at[pl.ds(wid * CH, CH)], dst_v)

    def body(j, carry):
        pltpu.async_copy(ones_v, dS_sh.at[src_v.at[j]], sem, add=True)
        pltpu.async_copy(ones_v, dD_sh.at[dst_v.at[j]], sem, add=True)

        @pl.when(j >= DEG_LAG)
        def _():
            pltpu.make_async_copy(ones_v, dS_sh.at[src_v.at[0]], sem).wait()
            pltpu.make_async_copy(ones_v, dD_sh.at[dst_v.at[0]], sem).wait()

        return carry

    lax.fori_loop(0, CH, body, 0)
    for _ in range(DEG_LAG):
        pltpu.make_async_copy(ones_v, dS_sh.at[src_v.at[0]], sem).wait()
        pltpu.make_async_copy(ones_v, dD_sh.at[dst_v.at[0]], sem).wait()
    plsc.subcore_barrier()
    pltpu.sync_copy(dS_sh.at[pl.ds(s * RPT, RPT)], out_h.at[c, 0, pl.ds(s * RPT, RPT)])
    pltpu.sync_copy(dD_sh.at[pl.ds(s * RPT, RPT)], out_h.at[c, 1, pl.ds(s * RPT, RPT)])


@functools.partial(
    pl.kernel,
    out_type=jax.ShapeDtypeStruct((NC, NPAD, D_H), jnp.float32),
    mesh=_mesh,
    compiler_params=_sc_params,
    scratch_types=[
        pltpu.VMEM((CH, BV), jnp.int32),
        pltpu.VMEM((CH, BV), jnp.int32),
        pltpu.VMEM((NB, BV, D_H), jnp.float32),
        pltpu.VMEM_SHARED((NPAD, D_H), jnp.float32),
        pltpu.SemaphoreType.DMA,
        pltpu.SemaphoreType.DMA,
    ],
)
def _sc_edge_agg(hn_h, src_h, dst_h, zrows_h, out_h, src_v, dst_v, rows_v, agg_sh,
                 gsem, ssem):
    c = lax.axis_index("c")
    s = lax.axis_index("s")
    wid = s * NC + c
    pltpu.sync_copy(zrows_h.at[pl.ds(s * RPT, RPT)], agg_sh.at[pl.ds(s * RPT, RPT)])
    plsc.subcore_barrier()
    pltpu.sync_copy(src_h.at[pl.ds(wid * CH, CH)], src_v)
    pltpu.sync_copy(dst_h.at[pl.ds(wid * CH, CH)], dst_v)

    for p in range(NB):
        pltpu.async_copy(hn_h.at[src_v.at[p]], rows_v.at[p], gsem)

    def body(j, carry):
        b = lax.rem(j, NB)

        @pl.when(jnp.logical_and(j >= 1, j + NB - 1 < CH))
        def _():
            pltpu.make_async_copy(rows_v.at[0], agg_sh.at[dst_v.at[0]], ssem).wait()
            pltpu.async_copy(
                hn_h.at[src_v.at[j + NB - 1]],
                rows_v.at[lax.rem(j + NB - 1, NB)],
                gsem,
            )

        pltpu.make_async_copy(hn_h.at[src_v.at[j]], rows_v.at[b], gsem).wait()
        pltpu.async_copy(rows_v.at[b], agg_sh.at[dst_v.at[j]], ssem, add=True)
        return carry

    lax.fori_loop(0, CH, body, 0)
    for _ in range(NB):
        pltpu.make_async_copy(rows_v.at[0], agg_sh.at[dst_v.at[0]], ssem).wait()
    plsc.subcore_barrier()
    pltpu.sync_copy(
        agg_sh.at[pl.ds(s * RPT, RPT)], out_h.at[c, pl.ds(s * RPT, RPT)]
    )


CR = 64
NCH_MID = RPT // CR


@functools.partial(
    pl.kernel,
    out_type=jax.ShapeDtypeStruct((NC, NPAD, D_H), jnp.float32),
    mesh=_mesh,
    compiler_params=_sc_params,
    scratch_types=[
        pltpu.VMEM((CH, BV), jnp.int32),
        pltpu.VMEM((CH, BV), jnp.int32),
        pltpu.VMEM((NB, BV, D_H), jnp.float32),
        pltpu.VMEM((CR, D_H), jnp.float32),
        pltpu.VMEM((CR, D_H), jnp.float32),
        pltpu.VMEM((CR, D_H), jnp.float32),
        pltpu.VMEM((CR, D_H), jnp.float32),
        pltpu.VMEM((CR, D_H), jnp.float32),
        pltpu.VMEM((D_H,), jnp.float32),
        pltpu.VMEM_SHARED((NPAD, D_H), jnp.float32),
        pltpu.VMEM_SHARED((NPAD, D_H), jnp.float32),
        pltpu.SemaphoreType.DMA,
        pltpu.SemaphoreType.DMA,
    ],
)
def _sc_mid_agg(aggp_h, nsrc_h, ndst_h, b1_h, src_h, dst_h, zrows_h, out_h,
                src_v, dst_v, rows_v, p0_v, p1_v, ns_v, nd_v, o_v, b1_v,
                agg_sh, hn_sh, gsem, ssem):
    c = lax.axis_index("c")
    s = lax.axis_index("s")
    wid = s * NC + c
    pltpu.sync_copy(zrows_h.at[pl.ds(s * RPT, RPT)], agg_sh.at[pl.ds(s * RPT, RPT)])
    pltpu.sync_copy(b1_h, b1_v)

    def mid_chunk(t, carry):
        r0 = s * RPT + t * CR
        pltpu.sync_copy(aggp_h.at[0, pl.ds(r0, CR)], p0_v)
        pltpu.sync_copy(aggp_h.at[1, pl.ds(r0, CR)], p1_v)
        pltpu.sync_copy(nsrc_h.at[pl.ds(r0, CR)], ns_v)
        pltpu.sync_copy(ndst_h.at[pl.ds(r0, CR)], nd_v)

        def row(i, carry2):
            a = p0_v[i] + p1_v[i]
            g = jnp.maximum(a * nd_v[i] + b1_v[...], 0.0) * ns_v[i]
            o_v[i] = g
            return carry2

        lax.fori_loop(0, CR, row, 0)
        pltpu.sync_copy(o_v, hn_sh.at[pl.ds(r0, CR)])
        return carry

    lax.fori_loop(0, NCH_MID, mid_chunk, 0)
    plsc.subcore_barrier()
    pltpu.sync_copy(src_h.at[pl.ds(wid * CH, CH)], src_v)
    pltpu.sync_copy(dst_h.at[pl.ds(wid * CH, CH)], dst_v)

    for p in range(NB):
        pltpu.async_copy(hn_sh.at[src_v.at[p]], rows_v.at[p], gsem)

    def body(j, carry):
        b = lax.rem(j, NB)

        @pl.when(jnp.logical_and(j >= 1, j + NB - 1 < CH))
        def _():
            pltpu.make_async_copy(rows_v.at[0], agg_sh.at[dst_v.at[0]], ssem).wait()
            pltpu.async_copy(
                hn_sh.at[src_v.at[j + NB - 1]],
                rows_v.at[lax.rem(j + NB - 1, NB)],
                gsem,
            )

        pltpu.make_async_copy(hn_sh.at[src_v.at[j]], rows_v.at[b], gsem).wait()
        pltpu.async_copy(rows_v.at[b], agg_sh.at[dst_v.at[j]], ssem, add=True)
        return carry

    lax.fori_loop(0, CH, body, 0)
    for _ in range(NB):
        pltpu.make_async_copy(rows_v.at[0], agg_sh.at[dst_v.at[0]], ssem).wait()
    plsc.subcore_barrier()
    pltpu.sync_copy(
        agg_sh.at[pl.ds(s * RPT, RPT)], out_h.at[c, pl.ds(s * RPT, RPT)]
    )


def _norms(degp_ref):
    d_src = degp_ref[0, 0] + degp_ref[1, 0]
    d_dst = degp_ref[0, 1] + degp_ref[1, 1]
    n_src = jnp.where(d_src > 0, lax.rsqrt(jnp.maximum(d_src, 1.0)), 0.0)
    n_dst = jnp.where(d_dst > 0, lax.rsqrt(jnp.maximum(d_dst, 1.0)), 0.0)
    return n_src, n_dst


def _tc_in_body(degp_ref, x_ref, w1_ref, o_ref, ns_ref, nd_ref):
    n_src, n_dst = _norms(degp_ref)
    h1 = jnp.dot(x_ref[...], w1_ref[...], preferred_element_type=jnp.float32)
    o_ref[...] = h1 * n_src
    ns_ref[...] = jnp.broadcast_to(n_src, (NPAD, D_H))
    nd_ref[...] = jnp.broadcast_to(n_dst, (NPAD, D_H))


def _tc_final_body(degp_ref, aggp_ref, w2_ref, b2_ref, o_ref):
    _, n_dst = _norms(degp_ref)
    t = (aggp_ref[0] + aggp_ref[1]) * n_dst
    o_ref[...] = (
        jnp.dot(t, w2_ref[...], preferred_element_type=jnp.float32) + b2_ref[...]
    )


def kernel(x, edge_index, W1, b1, W2, b2):
    src = edge_index[0].astype(jnp.int32)
    dst = edge_index[1].astype(jnp.int32)
    pad = N + jnp.arange(EPAD - E, dtype=jnp.int32) % (NPAD - N)
    src2d = jnp.concatenate([src, pad]).reshape(EPAD // BV, BV)
    dst2d = jnp.concatenate([dst, pad]).reshape(EPAD // BV, BV)
    xp = jnp.pad(x.astype(jnp.float32), ((0, NPAD - N), (0, 0)))
    znode = jnp.zeros((NPAD,), jnp.float32)
    zrows = jnp.zeros((NPAD, D_H), jnp.float32)
    b2r = b2.astype(jnp.float32).reshape(1, N_CLS)

    degp = _sc_degrees(src2d, dst2d, znode)
    degp4 = degp.reshape(NC, 2, NPAD, 1)

    hn1, nsrcb, ndstb = pl.pallas_call(
        _tc_in_body,
        out_shape=[
            jax.ShapeDtypeStruct((NPAD, D_H), jnp.float32),
            jax.ShapeDtypeStruct((NPAD, D_H), jnp.float32),
            jax.ShapeDtypeStruct((NPAD, D_H), jnp.float32),
        ],
    )(degp4, xp, W1.astype(jnp.float32))

    agg1p = _sc_edge_agg(hn1, src2d, dst2d, zrows)

    agg2p = _sc_mid_agg(
        agg1p, nsrcb, ndstb, b1.astype(jnp.float32), src2d, dst2d, zrows
    )

    outp = pl.pallas_call(
        _tc_final_body,
        out_shape=jax.ShapeDtypeStruct((NPAD, N_CLS), jnp.float32),
    )(degp4, agg2p, W2.astype(jnp.float32), b2r)

    return outp[:N]


# speedup vs baseline: 1.1169x; 1.1169x over previous
import functools

import jax
import jax.numpy as jnp
from jax import lax
from jax.experimental import pallas as pl
from jax.experimental.pallas import tpu as pltpu
from jax.experimental.pallas import tpu_sc as plsc

N = 10000
NPAD = 10240
D_IN = 128
D_H = 16
N_CLS = 7
E = 320000

NC = 2
NS = 16
NW = NC * NS
BV = 256
CH = 40
EPW = CH * BV
EPAD = NW * EPW
RPT = NPAD // NS
NB = 8
DEG_LAG = 8

_mesh = plsc.VectorSubcoreMesh(
    core_axis_name="c", subcore_axis_name="s", num_cores=NC, num_subcores=NS
)
_sc_params = pltpu.CompilerParams(use_tc_tiling_on_sc=False)


@functools.partial(
    pl.kernel,
    out_type=jax.ShapeDtypeStruct((NC, 2, NPAD), jnp.float32),
    mesh=_mesh,
    compiler_params=_sc_params,
    scratch_types=[
        pltpu.VMEM((CH, BV), jnp.int32),
        pltpu.VMEM((CH, BV), jnp.int32),
        pltpu.VMEM((BV,), jnp.float32),
        pltpu.VMEM_SHARED((NPAD,), jnp.float32),
        pltpu.VMEM_SHARED((NPAD,), jnp.float32),
        pltpu.SemaphoreType.DMA,
    ],
)
def _sc_degrees(src_h, dst_h, znode_h, out_h, src_v, dst_v, ones_v, dS_sh, dD_sh, sem):
    c = lax.axis_index("c")
    s = lax.axis_index("s")
    wid = s * NC + c
    for i in range(BV // 16):
        ones_v[pl.ds(i * 16, 16)] = jnp.ones((16,), jnp.float32)
    pltpu.sync_copy(znode_h.at[pl.ds(s * RPT, RPT)], dS_sh.at[pl.ds(s * RPT, RPT)])
    pltpu.sync_copy(znode_h.at[pl.ds(s * RPT, RPT)], dD_sh.at[pl.ds(s * RPT, RPT)])
    plsc.subcore_barrier()
    pltpu.sync_copy(src_h.at[pl.ds(wid * CH, CH)], src_v)
    pltpu.sync_copy(dst_h.at[pl.ds(wid * CH, CH)], dst_v)

    def body(j, carry):
        pltpu.async_copy(ones_v, dS_sh.at[src_v.at[j]], sem, add=True)
        pltpu.async_copy(ones_v, dD_sh.at[dst_v.at[j]], sem, add=True)

        @pl.when(j >= DEG_LAG)
        def _():
            pltpu.make_async_copy(ones_v, dS_sh.at[src_v.at[0]], sem).wait()
            pltpu.make_async_copy(ones_v, dD_sh.at[dst_v.at[0]], sem).wait()

        return carry

    lax.fori_loop(0, CH, body, 0)
    for _ in range(DEG_LAG):
        pltpu.make_async_copy(ones_v, dS_sh.at[src_v.at[0]], sem).wait()
        pltpu.make_async_copy(ones_v, dD_sh.at[dst_v.at[0]], sem).wait()
    plsc.subcore_barrier()
    pltpu.sync_copy(dS_sh.at[pl.ds(s * RPT, RPT)], out_h.at[c, 0, pl.ds(s * RPT, RPT)])
    pltpu.sync_copy(dD_sh.at[pl.ds(s * RPT, RPT)], out_h.at[c, 1, pl.ds(s * RPT, RPT)])


@functools.partial(
    pl.kernel,
    out_type=jax.ShapeDtypeStruct((NC, NPAD, D_H), jnp.float32),
    mesh=_mesh,
    compiler_params=_sc_params,
    scratch_types=[
        pltpu.VMEM((CH, BV), jnp.int32),
        pltpu.VMEM((CH, BV), jnp.int32),
        pltpu.VMEM((NB, BV, D_H), jnp.float32),
        pltpu.VMEM_SHARED((NPAD, D_H), jnp.float32),
        pltpu.SemaphoreType.DMA,
        pltpu.SemaphoreType.DMA,
    ],
)
def _sc_edge_agg(hn_h, src_h, dst_h, zrows_h, out_h, src_v, dst_v, rows_v, agg_sh,
                 gsem, ssem):
    c = lax.axis_index("c")
    s = lax.axis_index("s")
    wid = s * NC + c
    pltpu.sync_copy(zrows_h.at[pl.ds(s * RPT, RPT)], agg_sh.at[pl.ds(s * RPT, RPT)])
    plsc.subcore_barrier()
    pltpu.sync_copy(src_h.at[pl.ds(wid * CH, CH)], src_v)
    pltpu.sync_copy(dst_h.at[pl.ds(wid * CH, CH)], dst_v)

    for p in range(NB):
        pltpu.async_copy(hn_h.at[src_v.at[p]], rows_v.at[p], gsem)

    def body(j, carry):
        b = lax.rem(j, NB)

        @pl.when(jnp.logical_and(j >= 1, j + NB - 1 < CH))
        def _():
            pltpu.make_async_copy(rows_v.at[0], agg_sh.at[dst_v.at[0]], ssem).wait()
            pltpu.async_copy(
                hn_h.at[src_v.at[j + NB - 1]],
                rows_v.at[lax.rem(j + NB - 1, NB)],
                gsem,
            )

        pltpu.make_async_copy(hn_h.at[src_v.at[j]], rows_v.at[b], gsem).wait()
        pltpu.async_copy(rows_v.at[b], agg_sh.at[dst_v.at[j]], ssem, add=True)
        return carry

    lax.fori_loop(0, CH, body, 0)
    for _ in range(NB):
        pltpu.make_async_copy(rows_v.at[0], agg_sh.at[dst_v.at[0]], ssem).wait()
    plsc.subcore_barrier()
    pltpu.sync_copy(
        agg_sh.at[pl.ds(s * RPT, RPT)], out_h.at[c, pl.ds(s * RPT, RPT)]
    )


CR = 64
NCH_MID = RPT // CR


@functools.partial(
    pl.kernel,
    out_type=jax.ShapeDtypeStruct((NC, NPAD, D_H), jnp.float32),
    mesh=_mesh,
    compiler_params=_sc_params,
    scratch_types=[
        pltpu.VMEM((CH, BV), jnp.int32),
        pltpu.VMEM((CH, BV), jnp.int32),
        pltpu.VMEM((NB, BV, D_H), jnp.float32),
        pltpu.VMEM((2, CR, D_H), jnp.float32),
        pltpu.VMEM((2, CR, D_H), jnp.float32),
        pltpu.VMEM((2, CR, D_H), jnp.float32),
        pltpu.VMEM((2, CR, D_H), jnp.float32),
        pltpu.VMEM((2, CR, D_H), jnp.float32),
        pltpu.VMEM((D_H,), jnp.float32),
        pltpu.VMEM_SHARED((NPAD, D_H), jnp.float32),
        pltpu.VMEM_SHARED((NPAD, D_H), jnp.float32),
        pltpu.SemaphoreType.DMA,
        pltpu.SemaphoreType.DMA,
        pltpu.SemaphoreType.DMA,
        pltpu.SemaphoreType.DMA,
    ],
)
def _sc_mid_agg(aggp_h, nsrc_h, ndst_h, b1_h, src_h, dst_h, zrows_h, out_h,
                src_v, dst_v, rows_v, p0_v, p1_v, ns_v, nd_v, o_v, b1_v,
                agg_sh, hn_sh, gsem, ssem, isem, osem):
    c = lax.axis_index("c")
    s = lax.axis_index("s")
    wid = s * NC + c
    pltpu.sync_copy(zrows_h.at[pl.ds(s * RPT, RPT)], agg_sh.at[pl.ds(s * RPT, RPT)])
    pltpu.sync_copy(b1_h, b1_v)

    def issue_in(t, b):
        r0 = s * RPT + t * CR
        pltpu.async_copy(aggp_h.at[0, pl.ds(r0, CR)], p0_v.at[b], isem)
        pltpu.async_copy(aggp_h.at[1, pl.ds(r0, CR)], p1_v.at[b], isem)
        pltpu.async_copy(nsrc_h.at[pl.ds(r0, CR)], ns_v.at[b], isem)
        pltpu.async_copy(ndst_h.at[pl.ds(r0, CR)], nd_v.at[b], isem)

    def wait_in():
        for _ in range(4):
            pltpu.make_async_copy(
                aggp_h.at[0, pl.ds(0, CR)], p0_v.at[0], isem
            ).wait()

    def wait_out():
        pltpu.make_async_copy(
            o_v.at[0], hn_sh.at[pl.ds(0, CR)], osem
        ).wait()

    issue_in(0, 0)

    def mid_chunk(t, carry):
        b = lax.rem(t, 2)
        wait_in()

        @pl.when(t + 1 < NCH_MID)
        def _():
            issue_in(t + 1, 1 - b)

        @pl.when(t >= 2)
        def _():
            wait_out()

        def row(i, carry2):
            a = p0_v[b, i] + p1_v[b, i]
            g = jnp.maximum(a * nd_v[b, i] + b1_v[...], 0.0) * ns_v[b, i]
            o_v[b, i] = g
            return carry2

        lax.fori_loop(0, CR, row, 0)
        r0 = s * RPT + t * CR
        pltpu.async_copy(o_v.at[b], hn_sh.at[pl.ds(r0, CR)], osem)
        return carry

    lax.fori_loop(0, NCH_MID, mid_chunk, 0)
    for _ in range(2):
        wait_out()
    plsc.subcore_barrier()
    pltpu.sync_copy(src_h.at[pl.ds(wid * CH, CH)], src_v)
    pltpu.sync_copy(dst_h.at[pl.ds(wid * CH, CH)], dst_v)

    for p in range(NB):
        pltpu.async_copy(hn_sh.at[src_v.at[p]], rows_v.at[p], gsem)

    def body(j, carry):
        b = lax.rem(j, NB)

        @pl.when(jnp.logical_and(j >= 1, j + NB - 1 < CH))
        def _():
            pltpu.make_async_copy(rows_v.at[0], agg_sh.at[dst_v.at[0]], ssem).wait()
            pltpu.async_copy(
                hn_sh.at[src_v.at[j + NB - 1]],
                rows_v.at[lax.rem(j + NB - 1, NB)],
                gsem,
            )

        pltpu.make_async_copy(hn_sh.at[src_v.at[j]], rows_v.at[b], gsem).wait()
        pltpu.async_copy(rows_v.at[b], agg_sh.at[dst_v.at[j]], ssem, add=True)
        return carry

    lax.fori_loop(0, CH, body, 0)
    for _ in range(NB):
        pltpu.make_async_copy(rows_v.at[0], agg_sh.at[dst_v.at[0]], ssem).wait()
    plsc.subcore_barrier()
    pltpu.sync_copy(
        agg_sh.at[pl.ds(s * RPT, RPT)], out_h.at[c, pl.ds(s * RPT, RPT)]
    )


def _norms(degp_ref):
    d_src = degp_ref[0, 0] + degp_ref[1, 0]
    d_dst = degp_ref[0, 1] + degp_ref[1, 1]
    n_src = jnp.where(d_src > 0, lax.rsqrt(jnp.maximum(d_src, 1.0)), 0.0)
    n_dst = jnp.where(d_dst > 0, lax.rsqrt(jnp.maximum(d_dst, 1.0)), 0.0)
    return n_src, n_dst


def _tc_in_body(degp_ref, x_ref, w1_ref, o_ref, ns_ref, nd_ref):
    n_src, n_dst = _norms(degp_ref)
    h1 = jnp.dot(x_ref[...], w1_ref[...], preferred_element_type=jnp.float32)
    o_ref[...] = h1 * n_src
    ns_ref[...] = jnp.broadcast_to(n_src, (NPAD, D_H))
    nd_ref[...] = jnp.broadcast_to(n_dst, (NPAD, D_H))


def _tc_final_body(degp_ref, aggp_ref, w2_ref, b2_ref, o_ref):
    _, n_dst = _norms(degp_ref)
    t = (aggp_ref[0] + aggp_ref[1]) * n_dst
    o_ref[...] = (
        jnp.dot(t, w2_ref[...], preferred_element_type=jnp.float32) + b2_ref[...]
    )


def kernel(x, edge_index, W1, b1, W2, b2):
    src = edge_index[0].astype(jnp.int32)
    dst = edge_index[1].astype(jnp.int32)
    pad = N + jnp.arange(EPAD - E, dtype=jnp.int32) % (NPAD - N)
    src2d = jnp.concatenate([src, pad]).reshape(EPAD // BV, BV)
    dst2d = jnp.concatenate([dst, pad]).reshape(EPAD // BV, BV)
    xp = jnp.pad(x.astype(jnp.float32), ((0, NPAD - N), (0, 0)))
    znode = jnp.zeros((NPAD,), jnp.float32)
    zrows = jnp.zeros((NPAD, D_H), jnp.float32)
    b2r = b2.astype(jnp.float32).reshape(1, N_CLS)

    degp = _sc_degrees(src2d, dst2d, znode)
    degp4 = degp.reshape(NC, 2, NPAD, 1)

    hn1, nsrcb, ndstb = pl.pallas_call(
        _tc_in_body,
        out_shape=[
            jax.ShapeDtypeStruct((NPAD, D_H), jnp.float32),
            jax.ShapeDtypeStruct((NPAD, D_H), jnp.float32),
            jax.ShapeDtypeStruct((NPAD, D_H), jnp.float32),
        ],
    )(degp4, xp, W1.astype(jnp.float32))

    agg1p = _sc_edge_agg(hn1, src2d, dst2d, zrows)

    agg2p = _sc_mid_agg(
        agg1p, nsrcb, ndstb, b1.astype(jnp.float32), src2d, dst2d, zrows
    )

    outp = pl.pallas_call(
        _tc_final_body,
        out_shape=jax.ShapeDtypeStruct((NPAD, N_CLS), jnp.float32),
    )(degp4, agg2p, W2.astype(jnp.float32), b2r)

    return outp[:N]
